# baseline (speedup 1.0000x reference)
import functools

import jax
import jax.numpy as jnp
from jax import lax
from jax.experimental import pallas as pl
from jax.experimental.pallas import tpu as pltpu
from jax.experimental.pallas import tpu_sc as plsc

_B = 16384
_D = 128
_NC, _NS, _L = 2, 16, 16
_NW = _NC * _NS
_PW = _B // _NW
_NCH = 8
_CH = _PW // _NCH

_LN2 = 0.6931471805599453
_LG = (1.4425170337700015, -0.7178983812694636, 0.45689417400049853,
       -0.2773650501443504, 0.1219141394424037, -0.026066301477648768)


def _sc_partials(nodes_r, labels_r, w_emb, v_emb):
    mesh = plsc.VectorSubcoreMesh(core_axis_name="c", subcore_axis_name="s")

    @functools.partial(
        pl.kernel,
        out_type=jax.ShapeDtypeStruct((_NW, _L), jnp.float32),
        mesh=mesh,
        compiler_params=pltpu.CompilerParams(needs_layout_passes=False),
        scratch_types=[
            pltpu.VMEM((_PW,), jnp.int32),
            pltpu.VMEM((_PW,), jnp.int32),
            pltpu.VMEM((_PW,), jnp.float32),
            pltpu.VMEM((_CH, _D), jnp.float32),
            pltpu.VMEM((_CH, _D), jnp.float32),
            pltpu.VMEM((_CH, _D), jnp.float32),
            pltpu.VMEM((_CH, _D), jnp.float32),
            pltpu.VMEM((_PW,), jnp.float32),
            pltpu.VMEM(((_CH // _L) * _L * 17,), jnp.float32),
            pltpu.VMEM((_L,), jnp.float32),
            pltpu.SemaphoreType.DMA,
            pltpu.SemaphoreType.DMA,
            pltpu.SemaphoreType.DMA,
            pltpu.SemaphoreType.DMA,
        ],
    )
    def k(n0h, n1h, lbh, wh, vh, outh,
          idx0, idx1, lb_v, wA, wB, vA, vB, zbuf, stg, prow,
          swA, swB, svA, svB):
        wid = lax.axis_index("s") * _NC + lax.axis_index("c")
        pltpu.sync_copy(n0h.at[wid], idx0)
        pltpu.sync_copy(n1h.at[wid], idx1)
        pltpu.sync_copy(lbh.at[pl.ds(wid * _PW, _PW)], lb_v)

        iota16 = lax.iota(jnp.int32, _L)

        wb, vb = [wA, wB], [vA, vB]
        sw, sv = [swA, swB], [svA, svB]
        pend = [None, None]

        def start(c):
            p = c & 1
            cw = pltpu.async_copy(
                wh.at[idx0.at[pl.ds(c * _CH, _CH)]], wb[p], sw[p])
            cv = pltpu.async_copy(
                vh.at[idx1.at[pl.ds(c * _CH, _CH)]], vb[p], sv[p])
            pend[p] = (cw, cv)

        start(0)
        for c in range(_NCH):
            p = c & 1
            if c + 1 < _NCH:
                start(c + 1)
            cw, cv = pend[p]
            cw.wait()
            cv.wait()

            @plsc.parallel_loop(0, _CH // _L)
            def gbody(it, _p=p, _c=c):
                base_row = it * _L
                off = it * (_L * 17)

                def load_pair(row):
                    vals = []
                    for u in range(_D // _L):
                        vals.append(wb[_p][row, pl.ds(u * _L, _L)])
                        vals.append(vb[_p][row, pl.ds(u * _L, _L)])
                    return vals

                def reduce_store(vals, i):
                    ps = [vals[2 * u] * vals[2 * u + 1]
                          for u in range(_D // _L)]
                    while len(ps) > 1:
                        ps = [a + b for a, b in zip(ps[::2], ps[1::2])]
                    plsc.store_scatter(stg, [iota16 * 17 + (off + i)], ps[0])

                vals = load_pair(base_row)
                for i in range(_L):
                    nxt = load_pair(base_row + i + 1) if i + 1 < _L else None
                    reduce_store(vals, i)
                    vals = nxt
                qs = [stg[pl.ds(off + j * 17, _L)] for j in range(_L)]
                while len(qs) > 1:
                    qs = [a + b for a, b in zip(qs[::2], qs[1::2])]
                x = qs[0]

                t = lb_v[pl.ds(_c * _CH + it * _L, _L)]
                ax = jnp.abs(x)
                w = jnp.exp(-ax)
                y = 1.0 + w
                bits = lax.bitcast_convert_type(y, jnp.int32)
                e2 = lax.shift_right_logical(bits, 23) - 127
                m = lax.bitcast_convert_type(
                    (bits & 0x007FFFFF) | 0x3F800000, jnp.float32)
                d = m - 1.0
                pacc = jnp.float32(_LG[5])
                for cf in (_LG[4], _LG[3], _LG[2], _LG[1], _LG[0]):
                    pacc = pacc * d + cf
                log2y = e2.astype(jnp.float32) + d * pacc
                z = jnp.maximum(x, 0.0) - x * t + _LN2 * log2y
                zbuf[pl.ds(_c * _CH + it * _L, _L)] = z

        rs = [zbuf[pl.ds(j * _L, _L)] for j in range(_PW // _L)]
        while len(rs) > 1:
            rs = [a + b for a, b in zip(rs[::2], rs[1::2])]
        prow[...] = rs[0]
        pltpu.sync_copy(prow, outh.at[wid])

    return k(nodes_r[0], nodes_r[1], labels_r, w_emb, v_emb)


def _final_mean(partials):
    def body(p_ref, o_ref):
        ps = p_ref[...]
        col = jnp.sum(ps, axis=0, keepdims=True)
        o_ref[...] = jnp.sum(col, axis=1, keepdims=True) * (1.0 / _B)

    out = pl.pallas_call(
        body,
        out_shape=jax.ShapeDtypeStruct((1, 1), jnp.float32),
    )(partials)
    return out[0, 0]


def kernel(nodes, labels, w_emb, v_emb):
    n0 = nodes[:, 0].reshape(_NW, _PW)
    n1 = nodes[:, 1].reshape(_NW, _PW)
    partials = _sc_partials((n0, n1), labels, w_emb, v_emb)
    return _final_mean(partials)


# speedup vs baseline: 1.1351x; 1.1351x over previous
import functools

import jax
import jax.numpy as jnp
from jax import lax
from jax.experimental import pallas as pl
from jax.experimental.pallas import tpu as pltpu
from jax.experimental.pallas import tpu_sc as plsc

_B = 16384
_D = 128
_NC, _NS, _L = 2, 16, 16
_NW = _NC * _NS
_PW = _B // _NW
_NCH = 4
_CH = _PW // _NCH

_LN2 = 0.6931471805599453
_LG = (1.4425170337700015, -0.7178983812694636, 0.45689417400049853,
       -0.2773650501443504, 0.1219141394424037, -0.026066301477648768)


def _sc_partials(nodes_r, labels_r, w_emb, v_emb):
    mesh = plsc.VectorSubcoreMesh(core_axis_name="c", subcore_axis_name="s")

    @functools.partial(
        pl.kernel,
        out_type=jax.ShapeDtypeStruct((_NW, _L), jnp.float32),
        mesh=mesh,
        compiler_params=pltpu.CompilerParams(needs_layout_passes=False),
        scratch_types=[
            pltpu.VMEM((_PW,), jnp.int32),
            pltpu.VMEM((_PW,), jnp.int32),
            pltpu.VMEM((_PW,), jnp.float32),
            pltpu.VMEM((_CH, _D), jnp.float32),
            pltpu.VMEM((_CH, _D), jnp.float32),
            pltpu.VMEM((_CH, _D), jnp.float32),
            pltpu.VMEM((_CH, _D), jnp.float32),
            pltpu.VMEM((_PW,), jnp.float32),
            pltpu.VMEM(((_CH // _L) * _L * 17,), jnp.float32),
            pltpu.VMEM((_L,), jnp.float32),
            pltpu.SemaphoreType.DMA,
            pltpu.SemaphoreType.DMA,
            pltpu.SemaphoreType.DMA,
            pltpu.SemaphoreType.DMA,
        ],
    )
    def k(n0h, n1h, lbh, wh, vh, outh,
          idx0, idx1, lb_v, wA, wB, vA, vB, zbuf, stg, prow,
          swA, swB, svA, svB):
        wid = lax.axis_index("s") * _NC + lax.axis_index("c")
        pltpu.sync_copy(n0h.at[wid], idx0)
        pltpu.sync_copy(n1h.at[wid], idx1)
        pltpu.sync_copy(lbh.at[pl.ds(wid * _PW, _PW)], lb_v)

        iota16 = lax.iota(jnp.int32, _L)

        wb, vb = [wA, wB], [vA, vB]
        sw, sv = [swA, swB], [svA, svB]
        pend = [None, None]

        def start(c):
            p = c & 1
            cw = pltpu.async_copy(
                wh.at[idx0.at[pl.ds(c * _CH, _CH)]], wb[p], sw[p])
            cv = pltpu.async_copy(
                vh.at[idx1.at[pl.ds(c * _CH, _CH)]], vb[p], sv[p])
            pend[p] = (cw, cv)

        start(0)
        for c in range(_NCH):
            p = c & 1
            if c + 1 < _NCH:
                start(c + 1)
            cw, cv = pend[p]
            cw.wait()
            cv.wait()

            @plsc.parallel_loop(0, _CH // _L)
            def gbody(it, _p=p, _c=c):
                base_row = it * _L
                off = it * (_L * 17)

                def load_pair(row):
                    vals = []
                    for u in range(_D // _L):
                        vals.append(wb[_p][row, pl.ds(u * _L, _L)])
                        vals.append(vb[_p][row, pl.ds(u * _L, _L)])
                    return vals

                def reduce_store(vals, i):
                    ps = [vals[2 * u] * vals[2 * u + 1]
                          for u in range(_D // _L)]
                    while len(ps) > 1:
                        ps = [a + b for a, b in zip(ps[::2], ps[1::2])]
                    plsc.store_scatter(stg, [iota16 * 17 + (off + i)], ps[0])

                vals = load_pair(base_row)
                for i in range(_L):
                    nxt = load_pair(base_row + i + 1) if i + 1 < _L else None
                    reduce_store(vals, i)
                    vals = nxt
                qs = [stg[pl.ds(off + j * 17, _L)] for j in range(_L)]
                while len(qs) > 1:
                    qs = [a + b for a, b in zip(qs[::2], qs[1::2])]
                x = qs[0]

                t = lb_v[pl.ds(_c * _CH + it * _L, _L)]
                ax = jnp.abs(x)
                w = jnp.exp(-ax)
                y = 1.0 + w
                bits = lax.bitcast_convert_type(y, jnp.int32)
                e2 = lax.shift_right_logical(bits, 23) - 127
                m = lax.bitcast_convert_type(
                    (bits & 0x007FFFFF) | 0x3F800000, jnp.float32)
                d = m - 1.0
                pacc = jnp.float32(_LG[5])
                for cf in (_LG[4], _LG[3], _LG[2], _LG[1], _LG[0]):
                    pacc = pacc * d + cf
                log2y = e2.astype(jnp.float32) + d * pacc
                z = jnp.maximum(x, 0.0) - x * t + _LN2 * log2y
                zbuf[pl.ds(_c * _CH + it * _L, _L)] = z

        rs = [zbuf[pl.ds(j * _L, _L)] for j in range(_PW // _L)]
        while len(rs) > 1:
            rs = [a + b for a, b in zip(rs[::2], rs[1::2])]
        prow[...] = rs[0]
        pltpu.sync_copy(prow, outh.at[wid])

    return k(nodes_r[0], nodes_r[1], labels_r, w_emb, v_emb)


def _final_mean(partials):
    def body(p_ref, o_ref):
        ps = p_ref[...]
        col = jnp.sum(ps, axis=0, keepdims=True)
        o_ref[...] = jnp.sum(col, axis=1, keepdims=True) * (1.0 / _B)

    out = pl.pallas_call(
        body,
        out_shape=jax.ShapeDtypeStruct((1, 1), jnp.float32),
    )(partials)
    return out[0, 0]


def kernel(nodes, labels, w_emb, v_emb):
    n0 = nodes[:, 0].reshape(_NW, _PW)
    n1 = nodes[:, 1].reshape(_NW, _PW)
    partials = _sc_partials((n0, n1), labels, w_emb, v_emb)
    return _final_mean(partials)
